# baseline (speedup 1.0000x reference)
import functools

import jax
import jax.numpy as jnp
from jax import lax
from jax.experimental import pallas as pl
from jax.experimental.pallas import tpu as pltpu
from jax.experimental.pallas import tpu_sc as plsc

_D = 2048
_B = 4 * 8192
_SCALE = float(_D) ** 0.5
_NC, _NS, _L = 2, 16, 16
_NW = _NC * _NS
_BPW = _B // _NW
_C = 16
_NCHUNK = _BPW // _C

_mesh = plsc.VectorSubcoreMesh(core_axis_name="c", subcore_axis_name="s")


@functools.partial(
    pl.kernel,
    mesh=_mesh,
    out_type=jax.ShapeDtypeStruct((_B, _D), jnp.float32),
    scratch_types=[
        pltpu.VMEM((_BPW,), jnp.int32),
        pltpu.VMEM((_C, _D), jnp.float32),
        pltpu.SemaphoreType.DMA,
    ],
)
def _emb_lookup(ids_hbm, table_hbm, out_hbm, idx_v, rows_v, sem):
    wid = lax.axis_index("s") * _NC + lax.axis_index("c")
    base = wid * _BPW
    pltpu.sync_copy(ids_hbm.at[pl.ds(base, _BPW)], idx_v)

    def chunk_body(c, carry):
        pltpu.async_copy(
            table_hbm.at[idx_v.at[pl.ds(c * _C, _C)]], rows_v, sem
        ).wait()

        def row_body(i, carry2):
            for j in range(_D // _L):
                sl = pl.ds(j * _L, _L)
                rows_v[i, sl] = rows_v[i, sl] * _SCALE
            return carry2

        lax.fori_loop(0, _C, row_body, 0, unroll=False)

        pltpu.sync_copy(rows_v, out_hbm.at[pl.ds(base + c * _C, _C)])
        return carry

    lax.fori_loop(0, _NCHUNK, chunk_body, 0, unroll=False)


def kernel(input_ids, table):
    ids = input_ids.reshape(-1).astype(jnp.int32)
    out = _emb_lookup(ids, table)
    return out.reshape(*input_ids.shape, _D)


# speedup vs baseline: 1.5664x; 1.5664x over previous
import functools

import jax
import jax.numpy as jnp
from jax import lax
from jax.experimental import pallas as pl
from jax.experimental.pallas import tpu as pltpu
from jax.experimental.pallas import tpu_sc as plsc

_D = 2048
_B = 4 * 8192
_SCALE = float(_D) ** 0.5
_NC, _NS, _L = 2, 16, 16
_NW = _NC * _NS
_BPW = _B // _NW
_C = 16
_NCHUNK = _BPW // _C

_mesh = plsc.VectorSubcoreMesh(core_axis_name="c", subcore_axis_name="s")


@functools.partial(
    pl.kernel,
    mesh=_mesh,
    out_type=jax.ShapeDtypeStruct((_B, _D), jnp.float32),
    scratch_types=[
        pltpu.VMEM((_BPW,), jnp.int32),
        pltpu.VMEM((_C, _D), jnp.float32),
        pltpu.VMEM((_C, _D), jnp.float32),
        pltpu.SemaphoreType.DMA,
        pltpu.SemaphoreType.DMA,
        pltpu.SemaphoreType.DMA,
        pltpu.SemaphoreType.DMA,
    ],
)
def _emb_lookup(ids_hbm, table_hbm, out_hbm, idx_v, buf0, buf1,
                gsem0, gsem1, wsem0, wsem1):
    wid = lax.axis_index("s") * _NC + lax.axis_index("c")
    base = wid * _BPW
    pltpu.sync_copy(ids_hbm.at[pl.ds(base, _BPW)], idx_v)

    bufs = (buf0, buf1)
    gsems = (gsem0, gsem1)
    wsems = (wsem0, wsem1)

    def gather(c, buf, gsem):
        pltpu.async_copy(table_hbm.at[idx_v.at[pl.ds(c * _C, _C)]], buf, gsem)

    def wait_gather(c, buf, gsem):
        pltpu.make_async_copy(
            table_hbm.at[idx_v.at[pl.ds(c * _C, _C)]], buf, gsem
        ).wait()

    def write(c, buf, wsem):
        pltpu.async_copy(buf, out_hbm.at[pl.ds(base + c * _C, _C)], wsem)

    def wait_write(c, buf, wsem):
        pltpu.make_async_copy(buf, out_hbm.at[pl.ds(base + c * _C, _C)], wsem).wait()

    gather(0, buf0, gsem0)

    def pair_body(i, carry):
        g0 = i * 2
        for b in range(2):
            g = g0 + b
            buf, gsem, wsem = bufs[b], gsems[b], wsems[b]
            ob, ogsem, owsem = bufs[1 - b], gsems[1 - b], wsems[1 - b]

            @pl.when(g >= 1)
            def _():
                wait_write(g - 1, ob, owsem)

            @pl.when(g + 1 < _NCHUNK)
            def _():
                gather(g + 1, ob, ogsem)

            wait_gather(g, buf, gsem)

            def row_body(r, carry2):
                for j in range(_D // _L):
                    sl = pl.ds(j * _L, _L)
                    buf[r, sl] = buf[r, sl] * _SCALE
                return carry2

            lax.fori_loop(0, _C, row_body, 0, unroll=False)
            write(g, buf, wsem)
        return carry

    lax.fori_loop(0, _NCHUNK // 2, pair_body, 0, unroll=False)
    wait_write(_NCHUNK - 1, buf1, wsem1)


def kernel(input_ids, table):
    ids = input_ids.reshape(-1).astype(jnp.int32)
    out = _emb_lookup(ids, table)
    return out.reshape(*input_ids.shape, _D)
